# baseline (speedup 1.0000x reference)
import jax
import jax.numpy as jnp
from jax import lax
from jax.experimental import pallas as pl
from jax.experimental.pallas import tpu as pltpu
from jax.experimental.pallas import tpu_sc as plsc

NUM_FIELDS = 26
VOCAB = 100000
EMBED_DIM = 16
BATCH = 4096
DENSE_DIM = 13

NC = 2
NS = 16
NW = NC * NS
B_PER_W = BATCH // NW
LANES = 16


def _fm_body(disc_hbm, dense_hbm, table_hbm, w_hbm, b_hbm, out_hbm,
             idx_v, rows_v, dense_v, w_v, b_v, out_v, sem):
  wid = lax.axis_index("s") * NC + lax.axis_index("c")
  base = wid * B_PER_W

  pltpu.sync_copy(disc_hbm.at[:, pl.ds(base, B_PER_W)], idx_v)
  pltpu.sync_copy(dense_hbm.at[pl.ds(base, B_PER_W), :], dense_v)
  pltpu.sync_copy(w_hbm, w_v)
  pltpu.sync_copy(b_hbm, b_v)

  for f in range(NUM_FIELDS):
    for i in range(B_PER_W // LANES):
      sl = pl.ds(i * LANES, LANES)
      idx_v[f, sl] = idx_v[f, sl] + f * VOCAB

  copies = [
      pltpu.async_copy(table_hbm.at[idx_v.at[f]], rows_v.at[f], sem)
      for f in range(NUM_FIELDS)
  ]
  for c in copies:
    c.wait()

  bias = b_v[...]
  lane = lax.iota(jnp.int32, LANES)
  perms = [lane ^ sh for sh in (8, 4, 2, 1)]
  zero_lane = lane & 0

  def per_block(blk, _):
    acc = jnp.zeros((LANES,), jnp.float32)
    for l in range(LANES):
      j = blk * LANES + l
      row = dense_v[j, :]
      s = bias
      for d in range(DENSE_DIM):
        xd = row.at[zero_lane + d].get(mode="promise_in_bounds")
        s = s + xd * w_v[d]
      q = s * s
      for f in range(NUM_FIELDS):
        e = rows_v[f, j, :]
        s = s + e
        q = q + e * e
      t = s * s - q
      for p in perms:
        t = t + t.at[p].get(mode="promise_in_bounds")
      acc = jnp.where(lane == l, 0.5 * t, acc)
    out_v[pl.ds(blk * LANES, LANES)] = acc
    return 0

  lax.fori_loop(0, B_PER_W // LANES, per_block, 0)
  pltpu.sync_copy(out_v, out_hbm.at[pl.ds(base, B_PER_W)])


@jax.jit
def _fm_call(disc_t, dense_pad, table_flat, w_t, b):
  mesh = plsc.VectorSubcoreMesh(
      core_axis_name="c", subcore_axis_name="s", num_cores=NC, num_subcores=NS
  )
  return pl.kernel(
      _fm_body,
      out_type=jax.ShapeDtypeStruct((BATCH,), jnp.float32),
      mesh=mesh,
      compiler_params=pltpu.CompilerParams(use_tc_tiling_on_sc=False),
      scratch_types=[
          pltpu.VMEM((NUM_FIELDS, B_PER_W), jnp.int32),
          pltpu.VMEM((NUM_FIELDS, B_PER_W, EMBED_DIM), jnp.float32),
          pltpu.VMEM((B_PER_W, LANES), jnp.float32),
          pltpu.VMEM((DENSE_DIM, EMBED_DIM), jnp.float32),
          pltpu.VMEM((EMBED_DIM,), jnp.float32),
          pltpu.VMEM((B_PER_W,), jnp.float32),
          pltpu.SemaphoreType.DMA,
      ],
  )(disc_t, dense_pad, table_flat, w_t, b)


def kernel(dense_x, discrete_x, tables, W, b):
  disc_t = discrete_x.T
  dense_pad = jnp.pad(dense_x, ((0, 0), (0, LANES - DENSE_DIM)))
  table_flat = tables.reshape(NUM_FIELDS * VOCAB, EMBED_DIM)
  w_t = W.T
  return _fm_call(disc_t, dense_pad, table_flat, w_t, b)


# speedup vs baseline: 1.0013x; 1.0013x over previous
import jax
import jax.numpy as jnp
from jax import lax
from jax.experimental import pallas as pl
from jax.experimental.pallas import tpu as pltpu
from jax.experimental.pallas import tpu_sc as plsc

NUM_FIELDS = 26
VOCAB = 100000
EMBED_DIM = 16
BATCH = 4096
DENSE_DIM = 13

NC = 2
NS = 16
NW = NC * NS
B_PER_W = BATCH // NW
LANES = 16


def _fm_body(disc_hbm, dense_hbm, table_hbm, w_hbm, b_hbm, out_hbm,
             idx_v, rows_v, dense_v, w_v, b_v, out_v, sem):
  wid = lax.axis_index("s") * NC + lax.axis_index("c")
  base = wid * B_PER_W

  pltpu.sync_copy(disc_hbm.at[:, pl.ds(base, B_PER_W)], idx_v)
  pltpu.sync_copy(dense_hbm.at[pl.ds(base, B_PER_W), :], dense_v)
  pltpu.sync_copy(w_hbm, w_v)
  pltpu.sync_copy(b_hbm, b_v)

  copies = [
      pltpu.async_copy(table_hbm.at[f].at[idx_v.at[f]], rows_v.at[f], sem)
      for f in range(NUM_FIELDS)
  ]
  for c in copies:
    c.wait()

  bias = b_v[...]
  lane = lax.iota(jnp.int32, LANES)
  perms = [lane ^ sh for sh in (8, 4, 2, 1)]
  zero_lane = lane & 0

  def per_block(blk, _):
    acc = jnp.zeros((LANES,), jnp.float32)
    for l in range(LANES):
      j = blk * LANES + l
      row = dense_v[j, :]
      s = bias
      for d in range(DENSE_DIM):
        xd = row.at[zero_lane + d].get(mode="promise_in_bounds")
        s = s + xd * w_v[d]
      q = s * s
      for f in range(NUM_FIELDS):
        e = rows_v[f, j, :]
        s = s + e
        q = q + e * e
      t = s * s - q
      for p in perms:
        t = t + t.at[p].get(mode="promise_in_bounds")
      acc = jnp.where(lane == l, 0.5 * t, acc)
    out_v[pl.ds(blk * LANES, LANES)] = acc
    return 0

  lax.fori_loop(0, B_PER_W // LANES, per_block, 0)
  pltpu.sync_copy(out_v, out_hbm.at[pl.ds(base, B_PER_W)])


@jax.jit
def _fm_call(disc_t, dense_pad, table_flat, w_t, b):
  mesh = plsc.VectorSubcoreMesh(
      core_axis_name="c", subcore_axis_name="s", num_cores=NC, num_subcores=NS
  )
  return pl.kernel(
      _fm_body,
      out_type=jax.ShapeDtypeStruct((BATCH,), jnp.float32),
      mesh=mesh,
      compiler_params=pltpu.CompilerParams(use_tc_tiling_on_sc=False),
      scratch_types=[
          pltpu.VMEM((NUM_FIELDS, B_PER_W), jnp.int32),
          pltpu.VMEM((NUM_FIELDS, B_PER_W, EMBED_DIM), jnp.float32),
          pltpu.VMEM((B_PER_W, LANES), jnp.float32),
          pltpu.VMEM((DENSE_DIM, EMBED_DIM), jnp.float32),
          pltpu.VMEM((EMBED_DIM,), jnp.float32),
          pltpu.VMEM((B_PER_W,), jnp.float32),
          pltpu.SemaphoreType.DMA,
      ],
  )(disc_t, dense_pad, table_flat, w_t, b)


def kernel(dense_x, discrete_x, tables, W, b):
  disc_t = discrete_x.T
  dense_pad = jnp.pad(dense_x, ((0, 0), (0, LANES - DENSE_DIM)))
  w_t = W.T
  return _fm_call(disc_t, dense_pad, tables, w_t, b)


# speedup vs baseline: 3.1905x; 3.1864x over previous
import functools

import jax
import jax.numpy as jnp
from jax import lax
from jax.experimental import pallas as pl
from jax.experimental.pallas import tpu as pltpu
from jax.experimental.pallas import tpu_sc as plsc

NUM_FIELDS = 26
VOCAB = 100000
EMBED_DIM = 16
BATCH = 4096
DENSE_DIM = 13

NC = 2
NS = 16
NW = NC * NS
B_PER_W = BATCH // NW
LANES = 16
NBLK = B_PER_W // LANES


def _dense_body(x_ref, w_ref, b_ref, out_ref):
  out_ref[...] = (
      jax.lax.dot_general(
          w_ref[...], x_ref[...],
          dimension_numbers=(((1,), (1,)), ((), ())),
          preferred_element_type=jnp.float32,
      )
      + b_ref[...].reshape(EMBED_DIM, 1)
  )


def _fm_body(disc_hbm, dt_hbm, table_hbm, out_hbm,
             idx_v, cols_v, dt_v, out_v, sem):
  wid = lax.axis_index("s") * NC + lax.axis_index("c")
  base = wid * B_PER_W

  pltpu.sync_copy(disc_hbm.at[:, pl.ds(base, B_PER_W)], idx_v)
  pltpu.sync_copy(dt_hbm.at[:, pl.ds(base, B_PER_W)], dt_v)

  copies = []
  for f in range(NUM_FIELDS):
    for d in range(EMBED_DIM):
      copies.append(pltpu.async_copy(
          table_hbm.at[f].at[d].at[idx_v.at[f]],
          cols_v.at[f * EMBED_DIM + d], sem))
  for c in copies:
    c.wait()

  def per_block(i, _):
    sl = pl.ds(i * LANES, LANES)
    s = [dt_v[d, sl] for d in range(EMBED_DIM)]
    q = s[0] * s[0]
    for d in range(1, EMBED_DIM):
      q = q + s[d] * s[d]
    for f in range(NUM_FIELDS):
      for d in range(EMBED_DIM):
        c = cols_v[f * EMBED_DIM + d, sl]
        s[d] = s[d] + c
        q = q + c * c
    r = s[0] * s[0]
    for d in range(1, EMBED_DIM):
      r = r + s[d] * s[d]
    out_v[sl] = 0.5 * (r - q)
    return 0

  lax.fori_loop(0, NBLK, per_block, 0)
  pltpu.sync_copy(out_v, out_hbm.at[pl.ds(base, B_PER_W)])


@jax.jit
def _fm_call(dense_x, disc_t, tab_t, W, b):
  dt = pl.pallas_call(
      _dense_body,
      out_shape=jax.ShapeDtypeStruct((EMBED_DIM, BATCH), jnp.float32),
  )(dense_x, W, b)

  mesh = plsc.VectorSubcoreMesh(
      core_axis_name="c", subcore_axis_name="s", num_cores=NC, num_subcores=NS
  )
  return pl.kernel(
      _fm_body,
      out_type=jax.ShapeDtypeStruct((BATCH,), jnp.float32),
      mesh=mesh,
      compiler_params=pltpu.CompilerParams(use_tc_tiling_on_sc=False),
      scratch_types=[
          pltpu.VMEM((NUM_FIELDS, B_PER_W), jnp.int32),
          pltpu.VMEM((NUM_FIELDS * EMBED_DIM, B_PER_W), jnp.float32),
          pltpu.VMEM((EMBED_DIM, B_PER_W), jnp.float32),
          pltpu.VMEM((B_PER_W,), jnp.float32),
          pltpu.SemaphoreType.DMA,
      ],
  )(disc_t, dt, tab_t)


def kernel(dense_x, discrete_x, tables, W, b):
  disc_t = discrete_x.T
  tab_t = jnp.transpose(tables, (0, 2, 1))
  return _fm_call(dense_x, disc_t, tab_t, W, b)
